# baseline (speedup 1.0000x reference)
import functools

import jax
import jax.numpy as jnp
from jax import lax
from jax.experimental import pallas as pl
from jax.experimental.pallas import tpu as pltpu
from jax.experimental.pallas import tpu_sc as plsc

T, H, I, E, K = 2048, 1024, 512, 16, 2
BLK = 128
NB = (T * K) // BLK + E
NPAD = NB * BLK
NW = 32
TPW = T // NW


def _routing(topk_ids):
    ids = topk_ids.reshape(-1).astype(jnp.int32)
    oh = (ids[:, None] == jnp.arange(E, dtype=jnp.int32)[None, :]).astype(
        jnp.int32)
    incl = jnp.cumsum(oh, axis=0)
    counts = incl[-1]
    pcounts = ((counts + BLK - 1) // BLK) * BLK
    poffs = jnp.concatenate([jnp.zeros((1,), jnp.int32),
                             jnp.cumsum(pcounts)]).astype(jnp.int32)
    rank = jnp.sum(incl * oh, axis=1) - 1
    dest = jnp.sum(poffs[:E][None, :] * oh, axis=1) + rank
    blk_expert = jnp.clip(
        jnp.searchsorted(poffs, jnp.arange(NB, dtype=jnp.int32) * BLK,
                         side="right").astype(jnp.int32) - 1, 0, E - 1)
    return dest, blk_expert


def _sc_dispatch(x, dest0, dest1, w0, w1):
    mesh = plsc.VectorSubcoreMesh(core_axis_name="c", subcore_axis_name="s")

    @functools.partial(
        pl.kernel, mesh=mesh,
        out_type=(jax.ShapeDtypeStruct((NPAD, H), jnp.float32),
                  jax.ShapeDtypeStruct((NPAD,), jnp.float32)),
        scratch_types=[pltpu.VMEM((TPW,), jnp.int32),
                       pltpu.VMEM((TPW,), jnp.int32),
                       pltpu.VMEM((TPW,), jnp.float32),
                       pltpu.VMEM((TPW,), jnp.float32),
                       pltpu.VMEM((TPW, H), jnp.float32),
                       pltpu.SemaphoreType.DMA,
                       pltpu.SemaphoreType.DMA,
                       pltpu.SemaphoreType.DMA,
                       pltpu.SemaphoreType.DMA,
                       pltpu.SemaphoreType.DMA((5,))],
    )
    def k(x_hbm, d0_hbm, d1_hbm, w0_hbm, w1_hbm, xs_hbm, ws_hbm,
          i0_v, i1_v, w0_v, w1_v, rows_v, s0, s1, s2, s3, sl):
        wid = lax.axis_index("s") * 2 + lax.axis_index("c")
        base = wid * TPW
        la = pltpu.async_copy(d0_hbm.at[pl.ds(base, TPW)], i0_v, sl.at[0])
        lb = pltpu.async_copy(d1_hbm.at[pl.ds(base, TPW)], i1_v, sl.at[1])
        lc = pltpu.async_copy(w0_hbm.at[pl.ds(base, TPW)], w0_v, sl.at[2])
        ld = pltpu.async_copy(w1_hbm.at[pl.ds(base, TPW)], w1_v, sl.at[3])
        le = pltpu.async_copy(x_hbm.at[pl.ds(base, TPW)], rows_v, sl.at[4])
        la.wait()
        lb.wait()
        lc.wait()
        ld.wait()
        le.wait()
        c0 = pltpu.async_copy(rows_v, xs_hbm.at[i0_v], s0)
        c1 = pltpu.async_copy(rows_v, xs_hbm.at[i1_v], s1)
        c2 = pltpu.async_copy(w0_v, ws_hbm.at[i0_v], s2)
        c3 = pltpu.async_copy(w1_v, ws_hbm.at[i1_v], s3)
        c0.wait()
        c1.wait()
        c2.wait()
        c3.wait()

    return k(x, dest0, dest1, w0, w1)


def _tc_moe(xs, w13, w2, wsort, blk_expert):
    def body(be_ref, xs_ref, w13_hbm, w2_hbm, ws_ref, out_ref,
             w13_buf, w2_buf, st_ref, pend_ref, s13, s2):
        i = pl.program_id(0)
        e = be_ref[i]
        en = be_ref[jnp.minimum(i + 1, NB - 1)]

        @pl.when(i == 0)
        def _init():
            st_ref[0] = -1
            st_ref[1] = -1
            pend_ref[0] = 0
            pend_ref[1] = 0

        s = e % 2

        @pl.when(st_ref[s] != e)
        def _load():
            pltpu.make_async_copy(w13_hbm.at[e], w13_buf.at[s],
                                  s13.at[s]).start()
            pltpu.make_async_copy(w2_hbm.at[e], w2_buf.at[s],
                                  s2.at[s]).start()
            st_ref[s] = e
            pend_ref[s] = 1

        @pl.when(pend_ref[s] == 1)
        def _wait():
            pltpu.make_async_copy(w13_hbm.at[st_ref[s]], w13_buf.at[s],
                                  s13.at[s]).wait()
            pltpu.make_async_copy(w2_hbm.at[st_ref[s]], w2_buf.at[s],
                                  s2.at[s]).wait()
            pend_ref[s] = 0

        sn = en % 2

        @pl.when((en != e) & (sn != s) & (st_ref[sn] != en))
        def _prefetch():
            pltpu.make_async_copy(w13_hbm.at[en], w13_buf.at[sn],
                                  s13.at[sn]).start()
            pltpu.make_async_copy(w2_hbm.at[en], w2_buf.at[sn],
                                  s2.at[sn]).start()
            st_ref[sn] = en
            pend_ref[sn] = 1

        xsb = xs_ref[...].astype(jnp.bfloat16)
        h = lax.dot_general(xsb, w13_buf[s].astype(jnp.bfloat16),
                            (((1,), (0,)), ((), ())),
                            preferred_element_type=jnp.float32)
        gate = h[:, :I]
        up = h[:, I:]
        act = gate * jax.nn.sigmoid(gate) * up * ws_ref[...]
        o = lax.dot_general(act.astype(jnp.bfloat16),
                            w2_buf[s].astype(jnp.bfloat16),
                            (((1,), (0,)), ((), ())),
                            preferred_element_type=jnp.float32)
        out_ref[...] = o

    grid_spec = pltpu.PrefetchScalarGridSpec(
        num_scalar_prefetch=1,
        grid=(NB,),
        in_specs=[
            pl.BlockSpec((BLK, H), lambda b, be: (b, 0)),
            pl.BlockSpec(memory_space=pl.ANY),
            pl.BlockSpec(memory_space=pl.ANY),
            pl.BlockSpec((BLK, 1), lambda b, be: (b, 0)),
        ],
        out_specs=pl.BlockSpec((BLK, H), lambda b, be: (b, 0)),
        scratch_shapes=[
            pltpu.VMEM((2, H, 2 * I), jnp.float32),
            pltpu.VMEM((2, I, H), jnp.float32),
            pltpu.SMEM((2,), jnp.int32),
            pltpu.SMEM((2,), jnp.int32),
            pltpu.SemaphoreType.DMA((2,)),
            pltpu.SemaphoreType.DMA((2,)),
        ],
    )
    return pl.pallas_call(
        body, grid_spec=grid_spec,
        out_shape=jax.ShapeDtypeStruct((NPAD, H), jnp.float32),
    )(blk_expert, xs, w13, w2, wsort)


def _sc_combine(ys, dest):
    mesh = plsc.VectorSubcoreMesh(core_axis_name="c", subcore_axis_name="s")
    CH = 32
    nch = TPW // CH

    @functools.partial(
        pl.kernel, mesh=mesh,
        out_type=jax.ShapeDtypeStruct((T, H), jnp.float32),
        scratch_types=[pltpu.VMEM((2 * TPW,), jnp.int32),
                       pltpu.VMEM((2 * CH, H), jnp.float32),
                       pltpu.VMEM((CH, H), jnp.float32),
                       pltpu.SemaphoreType.DMA,
                       pltpu.SemaphoreType.DMA((2,))],
    )
    def k(ys_hbm, d_hbm, out_hbm, idx_v, pair_v, out_v, sg, sw):
        wid = lax.axis_index("s") * 2 + lax.axis_index("c")
        base = wid * TPW
        pltpu.sync_copy(d_hbm.at[pl.ds(2 * base, 2 * TPW)], idx_v)

        def add_pairs():
            def row(r, rc):
                @plsc.parallel_loop(0, H // 16, unroll=8)
                def col(c):
                    sl = pl.ds(c * 16, 16)
                    out_v[r, sl] = pair_v[2 * r, sl] + pair_v[2 * r + 1, sl]
                return rc

            lax.fori_loop(0, CH, row, 0)

        g = pltpu.async_copy(ys_hbm.at[idx_v.at[pl.ds(0, 2 * CH)]], pair_v, sg)
        g.wait()
        add_pairs()
        wb0 = pltpu.async_copy(out_v, out_hbm.at[pl.ds(base, CH)], sw.at[0])
        g = pltpu.async_copy(ys_hbm.at[idx_v.at[pl.ds(2 * CH, 2 * CH)]],
                             pair_v, sg)
        g.wait()
        wb0.wait()
        add_pairs()
        wb1 = pltpu.async_copy(out_v, out_hbm.at[pl.ds(base + CH, CH)],
                               sw.at[1])
        wb1.wait()

    return k(ys, dest)


def kernel(x, topk_weights, topk_ids, w13, w2):
    dest, blk_expert = _routing(topk_ids)
    dest2 = dest.reshape(T, K)
    w = topk_weights.astype(jnp.float32)
    xs, wsort = _sc_dispatch(x, dest2[:, 0], dest2[:, 1], w[:, 0], w[:, 1])
    ys = _tc_moe(xs, w13, w2, wsort.reshape(NPAD, 1), blk_expert)
    return _sc_combine(ys, dest)


# speedup vs baseline: 1.0872x; 1.0872x over previous
import functools

import jax
import jax.numpy as jnp
from jax import lax
from jax.experimental import pallas as pl
from jax.experimental.pallas import tpu as pltpu
from jax.experimental.pallas import tpu_sc as plsc

T, H, I, E, K = 2048, 1024, 512, 16, 2
BLK = 256
NB = (T * K) // BLK + E
NPAD = NB * BLK
NW = 32
TPW = T // NW


def _routing(topk_ids):
    ids = topk_ids.reshape(-1).astype(jnp.int32)
    oh = (ids[:, None] == jnp.arange(E, dtype=jnp.int32)[None, :]).astype(
        jnp.int32)
    incl = jnp.cumsum(oh, axis=0)
    counts = incl[-1]
    pcounts = ((counts + BLK - 1) // BLK) * BLK
    poffs = jnp.concatenate([jnp.zeros((1,), jnp.int32),
                             jnp.cumsum(pcounts)]).astype(jnp.int32)
    rank = jnp.sum(incl * oh, axis=1) - 1
    dest = jnp.sum(poffs[:E][None, :] * oh, axis=1) + rank
    blk_expert = jnp.clip(
        jnp.searchsorted(poffs, jnp.arange(NB, dtype=jnp.int32) * BLK,
                         side="right").astype(jnp.int32) - 1, 0, E - 1)
    return dest, blk_expert


def _sc_dispatch(x, dest0, dest1, w0, w1):
    mesh = plsc.VectorSubcoreMesh(core_axis_name="c", subcore_axis_name="s")

    @functools.partial(
        pl.kernel, mesh=mesh,
        out_type=(jax.ShapeDtypeStruct((NPAD, H), jnp.float32),
                  jax.ShapeDtypeStruct((NPAD,), jnp.float32)),
        scratch_types=[pltpu.VMEM((TPW,), jnp.int32),
                       pltpu.VMEM((TPW,), jnp.int32),
                       pltpu.VMEM((TPW,), jnp.float32),
                       pltpu.VMEM((TPW,), jnp.float32),
                       pltpu.VMEM((TPW, H), jnp.float32),
                       pltpu.SemaphoreType.DMA,
                       pltpu.SemaphoreType.DMA,
                       pltpu.SemaphoreType.DMA,
                       pltpu.SemaphoreType.DMA,
                       pltpu.SemaphoreType.DMA((5,))],
    )
    def k(x_hbm, d0_hbm, d1_hbm, w0_hbm, w1_hbm, xs_hbm, ws_hbm,
          i0_v, i1_v, w0_v, w1_v, rows_v, s0, s1, s2, s3, sl):
        wid = lax.axis_index("s") * 2 + lax.axis_index("c")
        base = wid * TPW
        la = pltpu.async_copy(d0_hbm.at[pl.ds(base, TPW)], i0_v, sl.at[0])
        lb = pltpu.async_copy(d1_hbm.at[pl.ds(base, TPW)], i1_v, sl.at[1])
        lc = pltpu.async_copy(w0_hbm.at[pl.ds(base, TPW)], w0_v, sl.at[2])
        ld = pltpu.async_copy(w1_hbm.at[pl.ds(base, TPW)], w1_v, sl.at[3])
        le = pltpu.async_copy(x_hbm.at[pl.ds(base, TPW)], rows_v, sl.at[4])
        la.wait()
        lb.wait()
        lc.wait()
        ld.wait()
        le.wait()
        c0 = pltpu.async_copy(rows_v, xs_hbm.at[i0_v], s0)
        c1 = pltpu.async_copy(rows_v, xs_hbm.at[i1_v], s1)
        c2 = pltpu.async_copy(w0_v, ws_hbm.at[i0_v], s2)
        c3 = pltpu.async_copy(w1_v, ws_hbm.at[i1_v], s3)
        c0.wait()
        c1.wait()
        c2.wait()
        c3.wait()

    return k(x, dest0, dest1, w0, w1)


def _tc_moe(xs, w13, w2, wsort, blk_expert):
    def body(be_ref, xs_ref, w13_ref, w2_ref, ws_ref, out_ref):
        xsb = xs_ref[...].astype(jnp.bfloat16)
        h = lax.dot_general(xsb, w13_ref[0].astype(jnp.bfloat16),
                            (((1,), (0,)), ((), ())),
                            preferred_element_type=jnp.float32)
        gate = h[:, :I]
        up = h[:, I:]
        act = gate * jax.nn.sigmoid(gate) * up
        o = lax.dot_general(act.astype(jnp.bfloat16),
                            w2_ref[0].astype(jnp.bfloat16),
                            (((1,), (0,)), ((), ())),
                            preferred_element_type=jnp.float32)
        out_ref[...] = o * ws_ref[...]

    grid_spec = pltpu.PrefetchScalarGridSpec(
        num_scalar_prefetch=1,
        grid=(NB,),
        in_specs=[
            pl.BlockSpec((BLK, H), lambda b, be: (b, 0)),
            pl.BlockSpec((1, H, 2 * I), lambda b, be: (be[b], 0, 0)),
            pl.BlockSpec((1, I, H), lambda b, be: (be[b], 0, 0)),
            pl.BlockSpec((BLK, 1), lambda b, be: (b, 0)),
        ],
        out_specs=pl.BlockSpec((BLK, H), lambda b, be: (b, 0)),
    )
    return pl.pallas_call(
        body, grid_spec=grid_spec,
        out_shape=jax.ShapeDtypeStruct((NPAD, H), jnp.float32),
    )(blk_expert, xs, w13, w2, wsort)


def _sc_combine(ys, dest):
    mesh = plsc.VectorSubcoreMesh(core_axis_name="c", subcore_axis_name="s")
    CH = 32
    nch = TPW // CH

    @functools.partial(
        pl.kernel, mesh=mesh,
        out_type=jax.ShapeDtypeStruct((T, H), jnp.float32),
        scratch_types=[pltpu.VMEM((2 * TPW,), jnp.int32),
                       pltpu.VMEM((2 * CH, H), jnp.float32),
                       pltpu.VMEM((CH, H), jnp.float32),
                       pltpu.SemaphoreType.DMA,
                       pltpu.SemaphoreType.DMA((2,))],
    )
    def k(ys_hbm, d_hbm, out_hbm, idx_v, pair_v, out_v, sg, sw):
        wid = lax.axis_index("s") * 2 + lax.axis_index("c")
        base = wid * TPW
        pltpu.sync_copy(d_hbm.at[pl.ds(2 * base, 2 * TPW)], idx_v)

        def add_pairs():
            def row(r, rc):
                @plsc.parallel_loop(0, H // 16, unroll=8)
                def col(c):
                    sl = pl.ds(c * 16, 16)
                    out_v[r, sl] = pair_v[2 * r, sl] + pair_v[2 * r + 1, sl]
                return rc

            lax.fori_loop(0, CH, row, 0)

        g = pltpu.async_copy(ys_hbm.at[idx_v.at[pl.ds(0, 2 * CH)]], pair_v, sg)
        g.wait()
        add_pairs()
        wb0 = pltpu.async_copy(out_v, out_hbm.at[pl.ds(base, CH)], sw.at[0])
        g = pltpu.async_copy(ys_hbm.at[idx_v.at[pl.ds(2 * CH, 2 * CH)]],
                             pair_v, sg)
        g.wait()
        wb0.wait()
        add_pairs()
        wb1 = pltpu.async_copy(out_v, out_hbm.at[pl.ds(base + CH, CH)],
                               sw.at[1])
        wb1.wait()

    return k(ys, dest)


def kernel(x, topk_weights, topk_ids, w13, w2):
    dest, blk_expert = _routing(topk_ids)
    dest2 = dest.reshape(T, K)
    w = topk_weights.astype(jnp.float32)
    xs, wsort = _sc_dispatch(x, dest2[:, 0], dest2[:, 1], w[:, 0], w[:, 1])
    ys = _tc_moe(xs, w13, w2, wsort.reshape(NPAD, 1), blk_expert)
    return _sc_combine(ys, dest)


# speedup vs baseline: 1.1116x; 1.0224x over previous
import functools

import jax
import jax.numpy as jnp
from jax import lax
from jax.experimental import pallas as pl
from jax.experimental.pallas import tpu as pltpu
from jax.experimental.pallas import tpu_sc as plsc

T, H, I, E, K = 2048, 1024, 512, 16, 2
BLK = 256
NB = (T * K) // BLK + E
NPAD = NB * BLK
NW = 32
TPW = T // NW


def _take(a, idx):
    return a.at[idx].get(mode="promise_in_bounds")


def _sc_counts(ids_flat):
    mesh = plsc.VectorSubcoreMesh(core_axis_name="c", subcore_axis_name="s")

    @functools.partial(
        pl.kernel, mesh=mesh,
        out_type=jax.ShapeDtypeStruct((NW, 16), jnp.int32),
        scratch_types=[pltpu.VMEM((2 * TPW,), jnp.int32),
                       pltpu.VMEM((16,), jnp.int32)],
    )
    def k(ids_hbm, pc_hbm, ids_v, pcrow_v):
        wid = lax.axis_index("s") * 2 + lax.axis_index("c")
        pltpu.sync_copy(ids_hbm.at[pl.ds(2 * TPW * wid, 2 * TPW)], ids_v)
        iota16 = lax.iota(jnp.int32, 16)
        zero16 = jnp.zeros((16,), jnp.int32)
        one16 = jnp.full((16,), 1, jnp.int32)
        vs = [ids_v[pl.ds(16 * j, 16)] for j in range(8)]

        def splat_sum(vv):
            s = vv
            for sh in (8, 4, 2, 1):
                s = s + _take(s, jnp.bitwise_and(iota16 + sh, 15))
            return s

        def ebody(e, pcrow):
            acc = zero16
            for v in vs:
                acc = acc + one16 - jnp.minimum(jnp.abs(v - e), 1)
            eq = one16 - jnp.minimum(jnp.abs(iota16 - e), 1)
            return pcrow + eq * splat_sum(acc)

        pcrow_v[...] = lax.fori_loop(0, E, ebody, zero16)
        pltpu.sync_copy(pcrow_v, pc_hbm.at[wid])

    return k(ids_flat)


def _sc_dispatch(x, ids_flat, pc, w0, w1):
    mesh = plsc.VectorSubcoreMesh(core_axis_name="c", subcore_axis_name="s")

    @functools.partial(
        pl.kernel, mesh=mesh,
        out_type=(jax.ShapeDtypeStruct((NPAD, H), jnp.float32),
                  jax.ShapeDtypeStruct((NPAD,), jnp.float32),
                  jax.ShapeDtypeStruct((T * K,), jnp.int32),
                  jax.ShapeDtypeStruct((NB,), jnp.int32)),
        scratch_types=[pltpu.VMEM((2 * TPW,), jnp.int32),
                       pltpu.VMEM((NW, 16), jnp.int32),
                       pltpu.VMEM((2 * TPW,), jnp.int32),
                       pltpu.VMEM((NB,), jnp.int32),
                       pltpu.VMEM((TPW,), jnp.int32),
                       pltpu.VMEM((TPW,), jnp.int32),
                       pltpu.VMEM((TPW,), jnp.float32),
                       pltpu.VMEM((TPW,), jnp.float32),
                       pltpu.VMEM((TPW, H), jnp.float32),
                       pltpu.SemaphoreType.DMA,
                       pltpu.SemaphoreType.DMA,
                       pltpu.SemaphoreType.DMA,
                       pltpu.SemaphoreType.DMA,
                       pltpu.SemaphoreType.DMA((5,))],
    )
    def k(x_hbm, ids_hbm, pc_hbm, w0_hbm, w1_hbm,
          xs_hbm, ws_hbm, dest_hbm, bexp_hbm,
          ids_v, pcs_v, destb_v, bexp_v,
          i0_v, i1_v, w0_v, w1_v, rows_v, s0, s1, s2, s3, sl):
        wid = lax.axis_index("s") * 2 + lax.axis_index("c")
        base = wid * TPW
        lc = pltpu.async_copy(w0_hbm.at[pl.ds(base, TPW)], w0_v, sl.at[2])
        ld = pltpu.async_copy(w1_hbm.at[pl.ds(base, TPW)], w1_v, sl.at[3])
        le = pltpu.async_copy(x_hbm.at[pl.ds(base, TPW)], rows_v, sl.at[4])
        pltpu.sync_copy(ids_hbm.at[pl.ds(2 * base, 2 * TPW)], ids_v)
        pltpu.sync_copy(pc_hbm, pcs_v)

        iota16 = lax.iota(jnp.int32, 16)
        zero16 = jnp.zeros((16,), jnp.int32)
        one16 = jnp.full((16,), 1, jnp.int32)
        full15 = jnp.full((16,), 15, jnp.int32)

        def cumsum16(vv):
            s = vv
            for sh in (1, 2, 4, 8):
                ge = jnp.minimum(jnp.maximum(iota16 - (sh - 1), 0), 1)
                s = s + ge * _take(s, jnp.maximum(iota16 - sh, 0))
            return s

        widv = jnp.full((16,), wid, jnp.int32)
        prefix = zero16
        total = zero16
        for j in range(NW):
            row = pcs_v[j]
            lt = jnp.minimum(jnp.maximum(widv - j, 0), 1)
            total = total + row
            prefix = prefix + lt * row
        pcounts = jnp.bitwise_and(total + (BLK - 1), -BLK)
        incl = cumsum16(pcounts)
        poffs = incl - pcounts

        run = prefix
        for j in range(8):
            v = ids_v[pl.ds(16 * j, 16)]
            base_v = _take(poffs, v) + _take(run, v)

            def ebody(e, c, _v=v):
                run_c, wcnt_c = c
                mi = one16 - jnp.minimum(jnp.abs(_v - e), 1)
                im = cumsum16(mi)
                wcnt_c = wcnt_c + mi * (im - 1)
                eq = one16 - jnp.minimum(jnp.abs(iota16 - e), 1)
                run_c = run_c + eq * _take(im, full15)
                return (run_c, wcnt_c)

            run, wcnt = lax.fori_loop(0, E, ebody, (run, zero16))
            destb_v[pl.ds(16 * j, 16)] = base_v + wcnt
        wd = pltpu.async_copy(destb_v, dest_hbm.at[pl.ds(2 * base, 2 * TPW)],
                              sl.at[0])

        lo = jnp.minimum(jnp.maximum(jnp.full((16,), 8, jnp.int32)
                                     - iota16, 0), 1)
        hi = one16 - lo
        for r in range(4):
            a1 = destb_v[pl.ds(32 * r, 16)]
            b1 = destb_v[pl.ds(32 * r + 16, 16)]
            ev = (lo * _take(a1, jnp.minimum(2 * iota16, 15))
                  + hi * _take(b1, jnp.maximum(2 * (iota16 - 8), 0)))
            od = (lo * _take(a1, jnp.minimum(2 * iota16 + 1, 15))
                  + hi * _take(b1, jnp.maximum(2 * (iota16 - 8) + 1, 0)))
            i0_v[pl.ds(16 * r, 16)] = ev
            i1_v[pl.ds(16 * r, 16)] = od

        for jj in range(NB // 16):
            bv = (iota16 + 16 * jj) * BLK

            def bbody(e, cnt, _bv=bv):
                ub = _take(incl, jnp.full((16,), e, jnp.int32))
                return cnt + jnp.minimum(jnp.maximum(_bv - ub + 1, 0), 1)

            bexp_v[pl.ds(16 * jj, 16)] = jnp.minimum(
                lax.fori_loop(0, E, bbody, zero16), E - 1)
        wb = pltpu.async_copy(bexp_v, bexp_hbm, sl.at[1])

        lc.wait()
        ld.wait()
        le.wait()
        c0 = pltpu.async_copy(rows_v, xs_hbm.at[i0_v], s0)
        c1 = pltpu.async_copy(rows_v, xs_hbm.at[i1_v], s1)
        c2 = pltpu.async_copy(w0_v, ws_hbm.at[i0_v], s2)
        c3 = pltpu.async_copy(w1_v, ws_hbm.at[i1_v], s3)
        c0.wait()
        c1.wait()
        c2.wait()
        c3.wait()
        wd.wait()
        wb.wait()

    return k(x, ids_flat, pc, w0, w1)


def _tc_moe(xs, w13, w2, wsort, blk_expert):
    def body(be_ref, xs_ref, w13_ref, w2_ref, ws_ref, out_ref):
        xsb = xs_ref[...].astype(jnp.bfloat16)
        h = lax.dot_general(xsb, w13_ref[0].astype(jnp.bfloat16),
                            (((1,), (0,)), ((), ())),
                            preferred_element_type=jnp.float32)
        gate = h[:, :I]
        up = h[:, I:]
        act = gate * jax.nn.sigmoid(gate) * up
        o = lax.dot_general(act.astype(jnp.bfloat16),
                            w2_ref[0].astype(jnp.bfloat16),
                            (((1,), (0,)), ((), ())),
                            preferred_element_type=jnp.float32)
        out_ref[...] = o * ws_ref[...]

    grid_spec = pltpu.PrefetchScalarGridSpec(
        num_scalar_prefetch=1,
        grid=(NB,),
        in_specs=[
            pl.BlockSpec((BLK, H), lambda b, be: (b, 0)),
            pl.BlockSpec((1, H, 2 * I), lambda b, be: (be[b], 0, 0)),
            pl.BlockSpec((1, I, H), lambda b, be: (be[b], 0, 0)),
            pl.BlockSpec((BLK, 1), lambda b, be: (b, 0)),
        ],
        out_specs=pl.BlockSpec((BLK, H), lambda b, be: (b, 0)),
    )
    return pl.pallas_call(
        body, grid_spec=grid_spec,
        out_shape=jax.ShapeDtypeStruct((NPAD, H), jnp.float32),
    )(blk_expert, xs, w13, w2, wsort)


def _sc_combine(ys, dest):
    mesh = plsc.VectorSubcoreMesh(core_axis_name="c", subcore_axis_name="s")
    CH = 32
    nch = TPW // CH

    @functools.partial(
        pl.kernel, mesh=mesh,
        out_type=jax.ShapeDtypeStruct((T, H), jnp.float32),
        scratch_types=[pltpu.VMEM((2 * TPW,), jnp.int32),
                       pltpu.VMEM((2 * CH, H), jnp.float32),
                       pltpu.VMEM((CH, H), jnp.float32),
                       pltpu.SemaphoreType.DMA,
                       pltpu.SemaphoreType.DMA((2,))],
    )
    def k(ys_hbm, d_hbm, out_hbm, idx_v, pair_v, out_v, sg, sw):
        wid = lax.axis_index("s") * 2 + lax.axis_index("c")
        base = wid * TPW
        pltpu.sync_copy(d_hbm.at[pl.ds(2 * base, 2 * TPW)], idx_v)

        def add_pairs():
            def row(r, rc):
                @plsc.parallel_loop(0, H // 16, unroll=8)
                def col(c):
                    sl = pl.ds(c * 16, 16)
                    out_v[r, sl] = pair_v[2 * r, sl] + pair_v[2 * r + 1, sl]
                return rc

            lax.fori_loop(0, CH, row, 0)

        g = pltpu.async_copy(ys_hbm.at[idx_v.at[pl.ds(0, 2 * CH)]], pair_v, sg)
        g.wait()
        add_pairs()
        wb0 = pltpu.async_copy(out_v, out_hbm.at[pl.ds(base, CH)], sw.at[0])
        g = pltpu.async_copy(ys_hbm.at[idx_v.at[pl.ds(2 * CH, 2 * CH)]],
                             pair_v, sg)
        g.wait()
        wb0.wait()
        add_pairs()
        wb1 = pltpu.async_copy(out_v, out_hbm.at[pl.ds(base + CH, CH)],
                               sw.at[1])
        wb1.wait()

    return k(ys, dest)


def kernel(x, topk_weights, topk_ids, w13, w2):
    ids_flat = topk_ids.reshape(-1).astype(jnp.int32)
    w = topk_weights.astype(jnp.float32)
    pc = _sc_counts(ids_flat)
    xs, wsort, dest, blk_expert = _sc_dispatch(x, ids_flat, pc,
                                               w[:, 0], w[:, 1])
    ys = _tc_moe(xs, w13, w2, wsort.reshape(NPAD, 1), blk_expert)
    return _sc_combine(ys, dest)
